# Initial kernel scaffold; baseline (speedup 1.0000x reference)
import functools

import jax
import jax.numpy as jnp
from jax import lax
from jax.experimental import pallas as pl
from jax.experimental.pallas import tpu as pltpu
from jax.experimental.pallas import tpu_sc as plsc

N_NODES = 10000
IN_DIM = 128
HIDDEN = 128
PROJ = 64
NUM_LAYERS = 4
NUM_GRAPHS = 64
N_EDGES = 320000

NC = 2
NS = 16
NW = NC * NS
CHUNK = 128
C_PER_W = 79
EDGES_PER_W = C_PER_W * CHUNK
E_PAD = NW * EDGES_PER_W
N_PAD = 10016
ZROWS_PER_S = N_PAD // NS
OROWS_PER_S = N_NODES // NS

_mesh = plsc.VectorSubcoreMesh(
    core_axis_name="c", subcore_axis_name="s", num_cores=NC, num_subcores=NS)


@functools.partial(
    pl.kernel,
    out_type=jax.ShapeDtypeStruct((NC, N_NODES, HIDDEN), jnp.float32),
    mesh=_mesh,
    scratch_types=[
        pltpu.VMEM((C_PER_W, CHUNK), jnp.int32),
        pltpu.VMEM((C_PER_W, CHUNK), jnp.int32),
        pltpu.VMEM((CHUNK, HIDDEN), jnp.float32),
        pltpu.VMEM_SHARED((N_PAD, HIDDEN), jnp.float32),
        pltpu.SemaphoreType.DMA,
    ],
)
def _sc_aggregate(src_hbm, dst_hbm, h_hbm, out_hbm, src_v, dst_v, rows_v,
                  agg_sh, sem):
    c = lax.axis_index("c")
    s = lax.axis_index("s")
    w = c * NS + s

    zero = jnp.zeros((16,), jnp.float32)

    def zrow(i, _):
        def zcol(j, _):
            rows_v[i, pl.ds(j * 16, 16)] = zero
            return 0
        return lax.fori_loop(0, HIDDEN // 16, zcol, 0)

    lax.fori_loop(0, CHUNK, zrow, 0)

    zbase = s * ZROWS_PER_S
    for k in range(ZROWS_PER_S // CHUNK):
        pltpu.sync_copy(rows_v, agg_sh.at[pl.ds(zbase + k * CHUNK, CHUNK)])
    rem = ZROWS_PER_S % CHUNK
    pltpu.sync_copy(rows_v.at[pl.ds(0, rem)],
                    agg_sh.at[pl.ds(zbase + (ZROWS_PER_S // CHUNK) * CHUNK, rem)])
    plsc.subcore_barrier()

    pltpu.sync_copy(src_hbm.at[w], src_v)
    pltpu.sync_copy(dst_hbm.at[w], dst_v)

    def chunk_body(i, _):
        pltpu.async_copy(h_hbm.at[src_v.at[i]], rows_v, sem).wait()
        pltpu.sync_copy(rows_v, agg_sh.at[dst_v.at[i]], add=True)
        return 0

    lax.fori_loop(0, C_PER_W, chunk_body, 0)
    plsc.subcore_barrier()

    obase = s * OROWS_PER_S
    pltpu.sync_copy(agg_sh.at[pl.ds(obase, OROWS_PER_S)],
                    out_hbm.at[c, pl.ds(obase, OROWS_PER_S)])


_MLP_BLK = 1000
_MLP_GRID = N_NODES // _MLP_BLK


def _mlp_body(relu_out, eps_ref, h_ref, a0_ref, a1_ref, w1_ref, b1_ref,
              w2_ref, b2_ref, o_ref):
    z = h_ref[:] * (1.0 + eps_ref[0, 0]) + a0_ref[0] + a1_ref[0]
    z = jnp.dot(z, w1_ref[:], preferred_element_type=jnp.float32) + b1_ref[:]
    z = jnp.maximum(z, 0.0)
    z = jnp.dot(z, w2_ref[:], preferred_element_type=jnp.float32) + b2_ref[:]
    if relu_out:
        z = jnp.maximum(z, 0.0)
    o_ref[:] = z


def _gin_mlp(h, agg2, w1, b1, w2, b2, eps_l, relu_out):
    return pl.pallas_call(
        functools.partial(_mlp_body, relu_out),
        grid=(_MLP_GRID,),
        in_specs=[
            pl.BlockSpec((1, 1), lambda i: (0, 0)),
            pl.BlockSpec((_MLP_BLK, HIDDEN), lambda i: (i, 0)),
            pl.BlockSpec((1, _MLP_BLK, HIDDEN), lambda i: (0, i, 0)),
            pl.BlockSpec((1, _MLP_BLK, HIDDEN), lambda i: (1, i, 0)),
            pl.BlockSpec((HIDDEN, HIDDEN), lambda i: (0, 0)),
            pl.BlockSpec((1, HIDDEN), lambda i: (0, 0)),
            pl.BlockSpec((HIDDEN, HIDDEN), lambda i: (0, 0)),
            pl.BlockSpec((1, HIDDEN), lambda i: (0, 0)),
        ],
        out_specs=pl.BlockSpec((_MLP_BLK, HIDDEN), lambda i: (i, 0)),
        out_shape=jax.ShapeDtypeStruct((N_NODES, HIDDEN), jnp.float32),
    )(eps_l, h, agg2, agg2, w1, b1.reshape(1, HIDDEN), w2,
      b2.reshape(1, HIDDEN))


def _pool_body(h_ref, batch_ref, wp1_ref, bp1_ref, wp2_ref, bp2_ref, o_ref,
               acc_ref, cnt_ref):
    i = pl.program_id(0)
    b = batch_ref[:]
    gid = lax.broadcasted_iota(jnp.int32, (NUM_GRAPHS, _MLP_BLK), 0)
    p = (gid == b).astype(jnp.float32)
    sums = jnp.dot(p, h_ref[:], preferred_element_type=jnp.float32)
    cnts = jnp.sum(p, axis=1, keepdims=True)

    @pl.when(i == 0)
    def _init():
        acc_ref[:] = sums
        cnt_ref[:] = cnts

    @pl.when(i > 0)
    def _accum():
        acc_ref[:] += sums
        cnt_ref[:] += cnts

    @pl.when(i == _MLP_GRID - 1)
    def _finish():
        zg = acc_ref[:] / jnp.maximum(cnt_ref[:], 1.0)
        z1 = jnp.dot(zg, wp1_ref[:], preferred_element_type=jnp.float32)
        z1 = jnp.maximum(z1 + bp1_ref[:], 0.0)
        o_ref[:] = (jnp.dot(z1, wp2_ref[:], preferred_element_type=jnp.float32)
                    + bp2_ref[:])


def _pool_project(h, batch_row, wp1, bp1, wp2, bp2):
    return pl.pallas_call(
        _pool_body,
        grid=(_MLP_GRID,),
        in_specs=[
            pl.BlockSpec((_MLP_BLK, HIDDEN), lambda i: (i, 0)),
            pl.BlockSpec((1, _MLP_BLK), lambda i: (0, i)),
            pl.BlockSpec((HIDDEN, HIDDEN), lambda i: (0, 0)),
            pl.BlockSpec((1, HIDDEN), lambda i: (0, 0)),
            pl.BlockSpec((HIDDEN, PROJ), lambda i: (0, 0)),
            pl.BlockSpec((1, PROJ), lambda i: (0, 0)),
        ],
        out_specs=pl.BlockSpec((NUM_GRAPHS, PROJ), lambda i: (0, 0)),
        out_shape=jax.ShapeDtypeStruct((NUM_GRAPHS, PROJ), jnp.float32),
        scratch_shapes=[
            pltpu.VMEM((NUM_GRAPHS, HIDDEN), jnp.float32),
            pltpu.VMEM((NUM_GRAPHS, 1), jnp.float32),
        ],
    )(h, batch_row, wp1, bp1.reshape(1, HIDDEN), wp2, bp2.reshape(1, PROJ))


def kernel(x, edge_index, batch, W1, b1, W2, b2, eps, Wp1, bp1, Wp2, bp2):
    src = edge_index[0].astype(jnp.int32)
    dst = edge_index[1].astype(jnp.int32)
    pad = E_PAD - N_EDGES
    src_p = jnp.concatenate([src, jnp.zeros((pad,), jnp.int32)])
    dst_p = jnp.concatenate([dst, jnp.full((pad,), N_NODES, jnp.int32)])
    src_p = src_p.reshape(NW, C_PER_W, CHUNK)
    dst_p = dst_p.reshape(NW, C_PER_W, CHUNK)

    h = x
    uts = []
    for l in range(NUM_LAYERS):
        agg2 = _sc_aggregate(src_p, dst_p, h)
        h = _gin_mlp(h, agg2, W1[l], b1[l], W2[l], b2[l],
                     eps[l].reshape(1, 1), relu_out=(l < NUM_LAYERS - 1))
        uts.append(h)

    H = uts[-1]
    z_proj = _pool_project(H, batch.astype(jnp.int32).reshape(1, N_NODES),
                           Wp1, bp1, Wp2, bp2)
    return (H, batch, z_proj) + tuple(uts)


# speedup vs baseline: 4.6424x; 4.6424x over previous
import functools

import jax
import jax.numpy as jnp
from jax import lax
from jax.experimental import pallas as pl
from jax.experimental.pallas import tpu as pltpu
from jax.experimental.pallas import tpu_sc as plsc

N_NODES = 10000
IN_DIM = 128
HIDDEN = 128
PROJ = 64
NUM_LAYERS = 4
NUM_GRAPHS = 64
N_EDGES = 320000

NC = 2
NS = 16
NW = NC * NS
CHUNK = 128
C_PER_W = 79
EDGES_PER_W = C_PER_W * CHUNK
E_PAD = NW * EDGES_PER_W
N_PAD = 10240
ROWS_PER_S = N_PAD // NS

@functools.lru_cache(maxsize=None)
def _make_sc_aggregate():
    mesh = plsc.VectorSubcoreMesh(
        core_axis_name="c", subcore_axis_name="s",
        num_cores=NC, num_subcores=NS)
    return pl.kernel(
        _sc_aggregate_body,
        out_type=jax.ShapeDtypeStruct((NC, N_PAD, HIDDEN), jnp.float32),
        mesh=mesh,
        scratch_types=[
            pltpu.VMEM((C_PER_W, CHUNK), jnp.int32),
            pltpu.VMEM((C_PER_W, CHUNK), jnp.int32),
            pltpu.VMEM((CHUNK, HIDDEN), jnp.float32),
            pltpu.VMEM_SHARED((N_PAD, HIDDEN), jnp.float32),
            pltpu.SemaphoreType.DMA,
        ],
    )


def _sc_aggregate_body(src_hbm, dst_hbm, h_hbm, out_hbm, src_v, dst_v, rows_v,
                       agg_sh, sem):
    c = lax.axis_index("c")
    s = lax.axis_index("s")
    w = c * NS + s

    zero = jnp.zeros((16,), jnp.float32)

    def zrow(i, _):
        def zcol(j, _):
            rows_v[i, pl.ds(j * 16, 16)] = zero
            return 0
        return lax.fori_loop(0, HIDDEN // 16, zcol, 0)

    lax.fori_loop(0, CHUNK, zrow, 0)

    zbase = s * ROWS_PER_S
    for k in range(ROWS_PER_S // CHUNK):
        pltpu.sync_copy(rows_v, agg_sh.at[pl.ds(zbase + k * CHUNK, CHUNK)])
    plsc.subcore_barrier()

    pltpu.sync_copy(src_hbm.at[w], src_v)
    pltpu.sync_copy(dst_hbm.at[w], dst_v)

    def chunk_body(i, _):
        pltpu.async_copy(h_hbm.at[src_v.at[i]], rows_v, sem).wait()
        pltpu.sync_copy(rows_v, agg_sh.at[dst_v.at[i]], add=True)
        return 0

    lax.fori_loop(0, C_PER_W, chunk_body, 0)
    plsc.subcore_barrier()

    obase = s * ROWS_PER_S
    pltpu.sync_copy(agg_sh.at[pl.ds(obase, ROWS_PER_S)],
                    out_hbm.at[c, pl.ds(obase, ROWS_PER_S)])


_MLP_BLK = 1000
_MLP_GRID = N_NODES // _MLP_BLK


def _mlp_body(relu_out, eps_ref, h_ref, a0_ref, a1_ref, w1_ref, b1_ref,
              w2_ref, b2_ref, o_ref):
    z = h_ref[:] * (1.0 + eps_ref[0, 0]) + a0_ref[0] + a1_ref[0]
    z = jnp.dot(z, w1_ref[:], preferred_element_type=jnp.float32) + b1_ref[:]
    z = jnp.maximum(z, 0.0)
    z = jnp.dot(z, w2_ref[:], preferred_element_type=jnp.float32) + b2_ref[:]
    if relu_out:
        z = jnp.maximum(z, 0.0)
    o_ref[:] = z


def _gin_mlp(h, agg2, w1, b1, w2, b2, eps_l, relu_out):
    return pl.pallas_call(
        functools.partial(_mlp_body, relu_out),
        grid=(_MLP_GRID,),
        in_specs=[
            pl.BlockSpec((1, 1), lambda i: (0, 0)),
            pl.BlockSpec((_MLP_BLK, HIDDEN), lambda i: (i, 0)),
            pl.BlockSpec((1, _MLP_BLK, HIDDEN), lambda i: (0, i, 0)),
            pl.BlockSpec((1, _MLP_BLK, HIDDEN), lambda i: (1, i, 0)),
            pl.BlockSpec((HIDDEN, HIDDEN), lambda i: (0, 0)),
            pl.BlockSpec((1, HIDDEN), lambda i: (0, 0)),
            pl.BlockSpec((HIDDEN, HIDDEN), lambda i: (0, 0)),
            pl.BlockSpec((1, HIDDEN), lambda i: (0, 0)),
        ],
        out_specs=pl.BlockSpec((_MLP_BLK, HIDDEN), lambda i: (i, 0)),
        out_shape=jax.ShapeDtypeStruct((N_NODES, HIDDEN), jnp.float32),
    )(eps_l, h, agg2, agg2, w1, b1.reshape(1, HIDDEN), w2,
      b2.reshape(1, HIDDEN))


def _pool_body(h_ref, batch_ref, wp1_ref, bp1_ref, wp2_ref, bp2_ref, o_ref,
               acc_ref, cnt_ref):
    i = pl.program_id(0)
    b = batch_ref[0]
    gid = lax.broadcasted_iota(jnp.int32, (NUM_GRAPHS, _MLP_BLK), 0)
    p = (gid == b).astype(jnp.float32)
    sums = jnp.dot(p, h_ref[:], preferred_element_type=jnp.float32)
    cnts = jnp.sum(p, axis=1, keepdims=True)

    @pl.when(i == 0)
    def _init():
        acc_ref[:] = sums
        cnt_ref[:] = cnts

    @pl.when(i > 0)
    def _accum():
        acc_ref[:] += sums
        cnt_ref[:] += cnts

    @pl.when(i == _MLP_GRID - 1)
    def _finish():
        zg = acc_ref[:] / jnp.maximum(cnt_ref[:], 1.0)
        z1 = jnp.dot(zg, wp1_ref[:], preferred_element_type=jnp.float32)
        z1 = jnp.maximum(z1 + bp1_ref[:], 0.0)
        o_ref[:] = (jnp.dot(z1, wp2_ref[:], preferred_element_type=jnp.float32)
                    + bp2_ref[:])


def _pool_project(h, batch_row, wp1, bp1, wp2, bp2):
    return pl.pallas_call(
        _pool_body,
        grid=(_MLP_GRID,),
        in_specs=[
            pl.BlockSpec((_MLP_BLK, HIDDEN), lambda i: (i, 0)),
            pl.BlockSpec((1, 1, _MLP_BLK), lambda i: (i, 0, 0)),
            pl.BlockSpec((HIDDEN, HIDDEN), lambda i: (0, 0)),
            pl.BlockSpec((1, HIDDEN), lambda i: (0, 0)),
            pl.BlockSpec((HIDDEN, PROJ), lambda i: (0, 0)),
            pl.BlockSpec((1, PROJ), lambda i: (0, 0)),
        ],
        out_specs=pl.BlockSpec((NUM_GRAPHS, PROJ), lambda i: (0, 0)),
        out_shape=jax.ShapeDtypeStruct((NUM_GRAPHS, PROJ), jnp.float32),
        scratch_shapes=[
            pltpu.VMEM((NUM_GRAPHS, HIDDEN), jnp.float32),
            pltpu.VMEM((NUM_GRAPHS, 1), jnp.float32),
        ],
    )(h, batch_row, wp1, bp1.reshape(1, HIDDEN), wp2, bp2.reshape(1, PROJ))


def kernel(x, edge_index, batch, W1, b1, W2, b2, eps, Wp1, bp1, Wp2, bp2):
    src = edge_index[0].astype(jnp.int32)
    dst = edge_index[1].astype(jnp.int32)
    pad = E_PAD - N_EDGES
    src_p = jnp.concatenate([src, jnp.zeros((pad,), jnp.int32)])
    dst_p = jnp.concatenate([dst, jnp.full((pad,), N_NODES, jnp.int32)])
    src_p = src_p.reshape(NW, C_PER_W, CHUNK)
    dst_p = dst_p.reshape(NW, C_PER_W, CHUNK)

    h = x
    uts = []
    for l in range(NUM_LAYERS):
        agg2 = _make_sc_aggregate()(src_p, dst_p, h)
        h = _gin_mlp(h, agg2, W1[l], b1[l], W2[l], b2[l],
                     eps[l].reshape(1, 1), relu_out=(l < NUM_LAYERS - 1))
        uts.append(h)

    H = uts[-1]
    z_proj = _pool_project(
        H, batch.astype(jnp.int32).reshape(_MLP_GRID, 1, _MLP_BLK),
        Wp1, bp1, Wp2, bp2)
    return (H, batch, z_proj) + tuple(uts)
